# baseline (speedup 1.0000x reference)
import functools

import jax
import jax.numpy as jnp
from jax import lax
from jax.experimental import pallas as pl
from jax.experimental.pallas import tpu as pltpu
from jax.experimental.pallas import tpu_sc as plsc

N_NODES = 100000
FEAT = 128
EMBED = 64
B = 4096
DEG = 32
K_SEL = 16

NC = 2
NS = 16
NW = NC * NS
RW = B // NW
NCHUNK = B * DEG // NW // 128

@functools.cache
def _sc_mesh():
    return plsc.VectorSubcoreMesh(core_axis_name="c", subcore_axis_name="s",
                                  num_cores=NC, num_subcores=NS)


SW = 16


def _scores_body(f_ref, cw_ref, cb_ref, o_ref):
    o_ref[...] = (
        jnp.dot(f_ref[...], cw_ref[...], preferred_element_type=jnp.float32)
        + cb_ref[0:1, :]
    )


def _all_scores(features, clf_w16, clf_b16):
    blk = 2000
    return pl.pallas_call(
        _scores_body,
        grid=(N_NODES // blk,),
        in_specs=[
            pl.BlockSpec((blk, FEAT), lambda i: (i, 0)),
            pl.BlockSpec((FEAT, SW), lambda i: (0, 0)),
            pl.BlockSpec((8, SW), lambda i: (0, 0)),
        ],
        out_specs=pl.BlockSpec((blk, SW), lambda i: (i, 0)),
        out_shape=jax.ShapeDtypeStruct((N_NODES, SW), jnp.float32),
    )(features, clf_w16, clf_b16)


def _gather_body(scores, features, nodes1d, nf1, nf2, nf3,
                 ns1, ns2, ns3, center, selff,
                 idxself, selfbuf, cbuf,
                 idx1d, nsb,
                 semself, sem1, sem2, sem3):
    w = lax.axis_index("s") * NC + lax.axis_index("c")
    nfs = (nf1, nf2, nf3)
    sems = (sem1, sem2, sem3)
    nsouts = (ns1, ns2, ns3)

    pltpu.sync_copy(nodes1d.at[pl.ds(w * RW, RW)], idxself)
    pltpu.async_copy(features.at[idxself], selfbuf, semself).wait()
    pltpu.async_copy(scores.at[idxself], cbuf, semself).wait()
    pltpu.sync_copy(selfbuf, selff.at[pl.ds(w * RW, RW)])
    pltpu.sync_copy(cbuf, center.at[pl.ds(w * RW, RW)])

    for rel in range(3):
        def step(r, carry, _rel=rel):
            base = w * NCHUNK * 128 + r * 128
            pltpu.sync_copy(nfs[_rel].at[pl.ds(base, 128)], idx1d)
            pltpu.async_copy(scores.at[idx1d], nsb, sems[_rel]).wait()
            pltpu.sync_copy(nsb, nsouts[_rel].at[pl.ds(base, 128)])
            return carry

        lax.fori_loop(0, NCHUNK, step, 0)


def _sc_gather(scores, features, nodes2d, nf1, nf2, nf3):
    fn = pl.kernel(
        _gather_body,
        out_type=[
            jax.ShapeDtypeStruct((B * DEG, SW), jnp.float32),
            jax.ShapeDtypeStruct((B * DEG, SW), jnp.float32),
            jax.ShapeDtypeStruct((B * DEG, SW), jnp.float32),
            jax.ShapeDtypeStruct((B, SW), jnp.float32),
            jax.ShapeDtypeStruct((B, FEAT), jnp.float32),
        ],
        mesh=_sc_mesh(),
        compiler_params=pltpu.CompilerParams(use_tc_tiling_on_sc=False),
        scratch_types=[
            pltpu.VMEM((RW,), jnp.int32),
            pltpu.VMEM((RW, FEAT), jnp.float32),
            pltpu.VMEM((RW, SW), jnp.float32),
            pltpu.VMEM((128,), jnp.int32),
            pltpu.VMEM((128, SW), jnp.float32),
            pltpu.SemaphoreType.DMA,
            pltpu.SemaphoreType.DMA,
            pltpu.SemaphoreType.DMA,
            pltpu.SemaphoreType.DMA,
        ],
    )
    return fn(scores, features, nodes2d, nf1, nf2, nf3)


def _select_body(c_ref, ns1_ref, ns2_ref, ns3_ref, nb1_ref, nb2_ref, nb3_ref,
                 s1_ref, s2_ref, s3_ref):
    blk = c_ref.shape[0]
    cs = c_ref[...]
    cc = jnp.broadcast_to(cs[:, None, :], (blk, DEG, SW)).reshape(
        blk, DEG * SW)
    mi = lax.broadcasted_iota(jnp.int32, (SW * DEG, DEG), 0) // SW
    mj = lax.broadcasted_iota(jnp.int32, (SW * DEG, DEG), 1)
    m = (mi == mj).astype(jnp.float32)
    ii = lax.broadcasted_iota(jnp.int32, (blk, DEG, DEG), 2)
    jj = lax.broadcasted_iota(jnp.int32, (blk, DEG, DEG), 1)
    for ns_ref, nb_ref, s_ref in ((ns1_ref, nb1_ref, s1_ref),
                                  (ns2_ref, nb2_ref, s2_ref),
                                  (ns3_ref, nb3_ref, s3_ref)):
        ad = jnp.abs(ns_ref[...] - cc)
        dist = lax.dot_general(ad, m, (((1,), (0,)), ((), ())),
                               precision=lax.Precision.HIGHEST)
        di = dist[:, None, :]
        dj = dist[:, :, None]
        beats = (di < dj) | ((di == dj) & (ii < jj))
        rank = jnp.sum(beats.astype(jnp.int32), axis=2)
        nb = nb_ref[...]
        rows = [jnp.sum(jnp.where(rank == k, nb, 0), axis=1)
                for k in range(K_SEL)]
        s_ref[0] = jnp.clip(jnp.stack(rows, axis=0), 0, N_NODES - 1)


def _tc_select(center, ns1, ns2, ns3, neigh1, neigh2, neigh3):
    blk = RW
    sel_shape = jax.ShapeDtypeStruct((NW, K_SEL, blk), jnp.int32)
    return pl.pallas_call(
        _select_body,
        grid=(NW,),
        in_specs=[
            pl.BlockSpec((blk, SW), lambda i: (i, 0)),
            pl.BlockSpec((blk, SW * DEG), lambda i: (i, 0)),
            pl.BlockSpec((blk, SW * DEG), lambda i: (i, 0)),
            pl.BlockSpec((blk, SW * DEG), lambda i: (i, 0)),
            pl.BlockSpec((blk, DEG), lambda i: (i, 0)),
            pl.BlockSpec((blk, DEG), lambda i: (i, 0)),
            pl.BlockSpec((blk, DEG), lambda i: (i, 0)),
        ],
        out_specs=[
            pl.BlockSpec((1, K_SEL, blk), lambda i: (i, 0, 0)),
            pl.BlockSpec((1, K_SEL, blk), lambda i: (i, 0, 0)),
            pl.BlockSpec((1, K_SEL, blk), lambda i: (i, 0, 0)),
        ],
        out_shape=[sel_shape, sel_shape, sel_shape],
    )(center, ns1, ns2, ns3, neigh1, neigh2, neigh3)


def _accum_body(features, sel1, sel2, sel3,
                agg1, agg2, agg3,
                idx1d, acc, sem):
    w = lax.axis_index("s") * NC + lax.axis_index("c")
    sels = (sel1, sel2, sel3)
    aggs = (agg1, agg2, agg3)

    for rel in range(3):
        def step(k, carry, _rel=rel):
            pltpu.sync_copy(
                sels[_rel].at[pl.ds(w * K_SEL * RW + k * RW, RW)], idx1d)
            pltpu.async_copy(features.at[idx1d], acc, sem).wait()
            pltpu.sync_copy(acc,
                            aggs[_rel].at[pl.ds(k * B + w * RW, RW)])
            return carry

        lax.fori_loop(0, K_SEL, step, 0)


def _sc_accum(features, sel1, sel2, sel3):
    agg_shape = jax.ShapeDtypeStruct((K_SEL * B, FEAT), jnp.float32)
    fn = pl.kernel(
        _accum_body,
        out_type=[agg_shape, agg_shape, agg_shape],
        mesh=_sc_mesh(),
        compiler_params=pltpu.CompilerParams(use_tc_tiling_on_sc=False),
        scratch_types=[
            pltpu.VMEM((RW,), jnp.int32),
            pltpu.VMEM((RW, FEAT), jnp.float32),
            pltpu.SemaphoreType.DMA,
        ],
    )
    return fn(features, sel1, sel2, sel3)


def _combine_body(s_ref, a1_ref, a2_ref, a3_ref,
                  w1_ref, w2_ref, w3_ref, wt_ref, o_ref):
    dn_t = (((0,), (1,)), ((), ()))
    dn_0 = (((0,), (0,)), ((), ()))
    w0 = wt_ref[0:FEAT, :]
    wr = [wt_ref[FEAT + i * EMBED:FEAT + (i + 1) * EMBED, :] for i in range(3)]
    u = lax.dot_general(w0, s_ref[...], dn_t,
                        preferred_element_type=jnp.float32)
    for i, a_ref in enumerate((a1_ref, a2_ref, a3_ref)):
        wi = (w1_ref, w2_ref, w3_ref)[i]
        asum = a_ref[0]
        for k in range(1, K_SEL):
            asum = asum + a_ref[k]
        rt = jax.nn.relu(lax.dot_general(wi[...], asum, dn_t,
                                         preferred_element_type=jnp.float32))
        u = u + lax.dot_general(wr[i], rt, dn_0,
                                preferred_element_type=jnp.float32)
    o_ref[...] = jax.nn.relu(u)


def _tc_combine(selff, agg1, agg2, agg3, w1s, w2s, w3s, weight):
    blk = 512
    return pl.pallas_call(
        _combine_body,
        grid=(B // blk,),
        in_specs=[
            pl.BlockSpec((blk, FEAT), lambda i: (i, 0)),
            pl.BlockSpec((K_SEL, blk, FEAT), lambda i: (0, i, 0)),
            pl.BlockSpec((K_SEL, blk, FEAT), lambda i: (0, i, 0)),
            pl.BlockSpec((K_SEL, blk, FEAT), lambda i: (0, i, 0)),
            pl.BlockSpec((FEAT, EMBED), lambda i: (0, 0)),
            pl.BlockSpec((FEAT, EMBED), lambda i: (0, 0)),
            pl.BlockSpec((FEAT, EMBED), lambda i: (0, 0)),
            pl.BlockSpec((FEAT + 3 * EMBED, EMBED), lambda i: (0, 0)),
        ],
        out_specs=pl.BlockSpec((EMBED, blk), lambda i: (0, i)),
        out_shape=jax.ShapeDtypeStruct((EMBED, B), jnp.float32),
    )(selff, agg1, agg2, agg3, w1s, w2s, w3s, weight)


def kernel(nodes, labels, features, train_pos, neigh_r1, neigh_r2, neigh_r3,
           weight, clf_w, clf_b, w1, w2, w3):
    del labels, train_pos
    clf_w16 = jnp.pad(clf_w, ((0, 0), (0, SW - 2)))
    clf_b16 = jnp.broadcast_to(jnp.pad(clf_b, (0, SW - 2)), (8, SW))
    scores = _all_scores(features, clf_w16, clf_b16)

    nodes1d = nodes.astype(jnp.int32)
    nf1 = neigh_r1.reshape(-1).astype(jnp.int32)
    nf2 = neigh_r2.reshape(-1).astype(jnp.int32)
    nf3 = neigh_r3.reshape(-1).astype(jnp.int32)
    ns1, ns2, ns3, center, selff = _sc_gather(
        scores, features, nodes1d, nf1, nf2, nf3)

    sel1, sel2, sel3 = _tc_select(
        center,
        ns1.reshape(B, SW * DEG), ns2.reshape(B, SW * DEG),
        ns3.reshape(B, SW * DEG),
        neigh_r1.astype(jnp.int32), neigh_r2.astype(jnp.int32),
        neigh_r3.astype(jnp.int32),
    )

    agg1, agg2, agg3 = _sc_accum(
        features,
        sel1.reshape(NW * K_SEL * RW),
        sel2.reshape(NW * K_SEL * RW),
        sel3.reshape(NW * K_SEL * RW),
    )

    inv = jnp.float32(1.0 / K_SEL)
    combined = _tc_combine(selff,
                           agg1.reshape(K_SEL, B, FEAT),
                           agg2.reshape(K_SEL, B, FEAT),
                           agg3.reshape(K_SEL, B, FEAT),
                           w1 * inv, w2 * inv, w3 * inv, weight)
    return combined, center[:, 0:2]
